# baseline (speedup 1.0000x reference)
import functools

import jax
import jax.numpy as jnp
from jax import lax
from jax.experimental import pallas as pl
from jax.experimental.pallas import tpu as pltpu
from jax.experimental.pallas import tpu_sc as plsc

B, S, D = 8, 2048, 8
D_MODEL = 256
BS = B * S

NC, NS, L = 2, 16, 16
NW = NC * NS
TW = BS // NW
C = 16
SLOTS = C * D
NCH = TW // C
NV = D_MODEL // L


def _sc_pooled(ids, tys, lens, key_table, index_table):
    mesh = plsc.VectorSubcoreMesh(core_axis_name="c", subcore_axis_name="s")

    @functools.partial(
        pl.kernel,
        out_type=jax.ShapeDtypeStruct((BS, D_MODEL), jnp.float32),
        mesh=mesh,
        scratch_types=[
            pltpu.VMEM((D_MODEL, D_MODEL), jnp.float32),
            pltpu.VMEM((SLOTS,), jnp.int32),
            pltpu.VMEM((SLOTS,), jnp.int32),
            pltpu.VMEM((SLOTS,), jnp.int32),
            pltpu.VMEM((SLOTS,), jnp.int32),
            pltpu.VMEM((SLOTS,), jnp.int32),
            pltpu.VMEM((SLOTS,), jnp.int32),
            pltpu.VMEM((SLOTS,), jnp.int32),
            pltpu.VMEM((SLOTS, D_MODEL), jnp.float32),
            pltpu.VMEM((C, D_MODEL), jnp.float32),
        ],
    )
    def k(ids_hbm, tys_hbm, lens_hbm, ktab_hbm, itab_hbm, out_hbm,
          tbl_v, ids_v, tys_v, lens_v, kidx_v, km_v, im_v, cid_v,
          krows_v, out_v):
        wid = lax.axis_index("s") * NC + lax.axis_index("c")
        pltpu.sync_copy(itab_hbm, tbl_v)
        pos = lax.rem(lax.iota(jnp.int32, 16), D)

        @pl.loop(0, NCH)
        def _(ch):
            tok0 = wid * TW + ch * C
            s0 = tok0 * D
            pltpu.sync_copy(ids_hbm.at[pl.ds(s0, SLOTS)], ids_v)
            pltpu.sync_copy(tys_hbm.at[pl.ds(s0, SLOTS)], tys_v)
            pltpu.sync_copy(lens_hbm.at[pl.ds(s0, SLOTS)], lens_v)

            for g in range(SLOTS // L):
                sl = pl.ds(g * L, L)
                idv = ids_v[sl]
                tyv = tys_v[sl]
                lnv = lens_v[sl]
                valid = pos < lnv
                km = jnp.where(valid & (tyv == 1), 1, 0)
                im = jnp.where(valid & (tyv == 2), 1, 0)
                kidx_v[sl] = jnp.where(km == 1, idv, 0)
                km_v[sl] = km
                im_v[sl] = im
                cid_v[sl] = jnp.minimum(idv, D_MODEL - 1)

            pltpu.sync_copy(ktab_hbm.at[kidx_v], krows_v)

            @pl.loop(0, C)
            def _(t):
                for v in range(NV):
                    out_v[t, pl.ds(v * L, L)] = jnp.zeros((L,), jnp.float32)

            @pl.loop(0, SLOTS // L)
            def _(g):
                kmv = km_v[pl.ds(g * L, L)]
                imv = im_v[pl.ds(g * L, L)]
                cidv = cid_v[pl.ds(g * L, L)]
                for dd in range(L):
                    s = g * L + dd
                    t = g * (L // D) + dd // D

                    @pl.when(kmv[dd] != 0)
                    def _():
                        for v in range(NV):
                            out_v[t, pl.ds(v * L, L)] += krows_v[s, pl.ds(v * L, L)]

                    @pl.when(imv[dd] != 0)
                    def _():
                        cid = cidv[dd]
                        for v in range(NV):
                            out_v[t, pl.ds(v * L, L)] += tbl_v[cid, pl.ds(v * L, L)]

            pltpu.sync_copy(out_v, out_hbm.at[pl.ds(tok0, C)])

    return k(ids, tys, lens, key_table, index_table)


@jax.jit
def kernel(path_types, path_ids, path_lengths, key_table, index_table):
    ids = path_ids.reshape(-1).astype(jnp.int32)
    tys = path_types.reshape(-1).astype(jnp.int32)
    lens = jnp.broadcast_to(
        path_lengths.astype(jnp.int32)[..., None], (B, S, D)
    ).reshape(-1)
    out = _sc_pooled(ids, tys, lens,
                     key_table.astype(jnp.float32),
                     index_table.astype(jnp.float32))
    return out.reshape(B, S, D_MODEL)
